# baseline (speedup 1.0000x reference)
import functools

import jax
import jax.numpy as jnp
from jax import lax
from jax.experimental import pallas as pl
from jax.experimental.pallas import tpu as pltpu
from jax.experimental.pallas import tpu_sc as plsc

_LATENT = 16
_NSTEPS = 2

_NC, _NS = 2, 16
_NW = _NC * _NS

_B, _N, _D = 4, 100000, 20
_BLK = 1024
_NBLK = 98
_NPB = _NBLK * _BLK
_PRB = 13312
_VPB = _PRB * 8
_ROWS_PER_W = _VPB // _NW
_CHUNK = 256
_NCHUNK = _ROWS_PER_W // _CHUNK
_IDX_W = 128
_IPC = _CHUNK // _IDX_W


def _tables_body(q_ref, w1_ref, t0_ref, t1_ref, t2_ref):
    outs = (t0_ref, t1_ref, t2_ref)
    qb = q_ref[...]
    for k in range(3):
        w1k = w1_ref[k * _D:(k + 1) * _D, :]
        m = lax.dot_general(qb, w1k, (((0,), (0,)), ((), ())),
                            preferred_element_type=jnp.float32)
        outs[k][:, :] = jnp.concatenate(
            [m[s * 128:(s + 1) * 128, :] for s in range(8)], axis=1)


@functools.cache
def _make_tables():
    return pl.pallas_call(
        _tables_body,
        grid=(_NBLK,),
        in_specs=[
            pl.BlockSpec((_D, _BLK), lambda i: (0, i)),
            pl.BlockSpec((3 * _D, _LATENT), lambda i: (0, 0)),
        ],
        out_specs=[pl.BlockSpec((128, 128), lambda i: (i, 0))] * 3,
        out_shape=[jax.ShapeDtypeStruct((_PRB, 128), jnp.float32)] * 3,
    )


def _gather_sum_body(p0, p1, p2, i0, i1, i2, out_hbm,
                     iv, rv, ov, sem):
    wid = lax.axis_index("s") * _NC + lax.axis_index("c")
    tabs = (p0, p1, p2)
    idxs = (i0, i1, i2)

    def issue(c, buf):
        irow = wid * (_ROWS_PER_W // _IDX_W) + c * _IPC
        for k in range(3):
            pltpu.sync_copy(idxs[k].at[pl.ds(irow, _IPC)], iv.at[buf, k])
        for j in range(_IPC):
            for k in range(3):
                pltpu.async_copy(tabs[k].at[iv.at[buf, k, j]],
                                 rv.at[buf, k, pl.ds(j * _IDX_W, _IDX_W)],
                                 sem)

    def drain(buf):
        for j in range(_IPC):
            for k in range(3):
                pltpu.make_async_copy(
                    tabs[k].at[iv.at[buf, k, j]],
                    rv.at[buf, k, pl.ds(j * _IDX_W, _IDX_W)], sem).wait()

    def sum_store(c, buf):
        def row_body(i, carry2):
            ov[i, :] = (rv[buf, 0, i, :] + rv[buf, 1, i, :]
                        + rv[buf, 2, i, :])
            return carry2

        lax.fori_loop(0, _CHUNK, row_body, 0, unroll=8)
        base = wid * _ROWS_PER_W + c * _CHUNK
        pltpu.sync_copy(ov, out_hbm.at[pl.ds(base, _CHUNK)])

    issue(0, 0)

    def outer(t, carry):
        c2 = 2 * t
        drain(0)
        issue(c2 + 1, 1)
        sum_store(c2, 0)
        drain(1)
        issue(c2 + 2, 0)
        sum_store(c2 + 1, 1)
        return carry

    lax.fori_loop(0, (_NCHUNK - 1) // 2, outer, 0)
    drain(0)
    sum_store(_NCHUNK - 1, 0)


@functools.cache
def _get_gather_sum():
    return pl.kernel(
        _gather_sum_body,
        out_type=jax.ShapeDtypeStruct((_VPB, _LATENT), jnp.float32),
        mesh=plsc.VectorSubcoreMesh(core_axis_name="c", subcore_axis_name="s",
                                    num_cores=_NC, num_subcores=_NS),
        scratch_types=[
            pltpu.VMEM((2, 3, _IPC, _IDX_W), jnp.int32),
            pltpu.VMEM((2, 3, _CHUNK, _LATENT), jnp.float32),
            pltpu.VMEM((_CHUNK, _LATENT), jnp.float32),
            pltpu.SemaphoreType.DMA,
        ],
        compiler_params=pltpu.CompilerParams(use_tc_tiling_on_sc=False),
    )


def _update_body(q_ref, g_ref, b1_ref, w2_ref, b2_ref, o_ref):
    gb = g_ref[...]
    mg = jnp.concatenate(
        [gb[:, s * 16:(s + 1) * 16] for s in range(8)], axis=0)
    h = jnp.tanh(mg + b1_ref[...])
    f = jnp.dot(h, w2_ref[...],
                preferred_element_type=jnp.float32) + b2_ref[...]
    o_ref[...] = q_ref[...] + jnp.transpose(f, (1, 0))


@functools.cache
def _make_update():
    return pl.pallas_call(
        _update_body,
        grid=(_NBLK,),
        in_specs=[
            pl.BlockSpec((_D, _BLK), lambda i: (0, i)),
            pl.BlockSpec((128, 128), lambda i: (i, 0)),
            pl.BlockSpec((1, _LATENT), lambda i: (0, 0)),
            pl.BlockSpec((_LATENT, _D), lambda i: (0, 0)),
            pl.BlockSpec((1, _D), lambda i: (0, 0)),
        ],
        out_specs=pl.BlockSpec((_D, _BLK), lambda i: (0, i)),
        out_shape=jax.ShapeDtypeStruct((_D, _N), jnp.float32),
    )


def kernel(inputs, W1, b1, W2, b2, neighbour_list):
    b, n, d = inputs.shape
    qbs = [jnp.transpose(inputs[bb], (1, 0)) for bb in range(_B)]

    padrows = jnp.broadcast_to(
        (jnp.arange(_NPB - n, dtype=jnp.int32) * 997 % n)[:, None],
        (_NPB - n, 3))
    nbr = jnp.concatenate([neighbour_list, padrows], axis=0)
    j = nbr.T
    q = j % _BLK
    vloc = (j // _BLK) * _BLK + (q % 128) * 8 + q // 128
    vloc = vloc.reshape(3, _NBLK, 8, 128).swapaxes(2, 3).reshape(3, _NPB)
    padv = jnp.broadcast_to(
        (jnp.arange(_VPB - _NPB, dtype=jnp.int32) * 1013 % _NPB)[None, :],
        (3, _VPB - _NPB))
    vloc = jnp.concatenate([vloc, padv], axis=1)
    idx = vloc.reshape(3, _VPB // _IDX_W, _IDX_W)
    i0, i1, i2 = idx[0], idx[1], idx[2]

    w2p = jnp.pad(W2, ((0, 0), (0, d - _LATENT)))
    b1r = b1.reshape(1, _LATENT)
    b2r = jnp.pad(b2, (0, d - _LATENT)).reshape(1, d)

    tables = _make_tables()
    upd = _make_update()
    gather = _get_gather_sum()

    for _ in range(_NSTEPS):
        ts = [tables(qbs[bb], W1) for bb in range(_B)]
        gs = [gather(t0.reshape(_VPB, _LATENT), t1.reshape(_VPB, _LATENT),
                     t2.reshape(_VPB, _LATENT), i0, i1, i2)
              for (t0, t1, t2) in ts]
        qbs = [upd(qbs[bb], gs[bb].reshape(_PRB, 128), b1r, w2p, b2r)
               for bb in range(_B)]
    return jnp.transpose(jnp.stack(qbs), (0, 2, 1))


# speedup vs baseline: 1.4275x; 1.4275x over previous
import functools

import jax
import jax.numpy as jnp
from jax import lax
from jax.experimental import pallas as pl
from jax.experimental.pallas import tpu as pltpu
from jax.experimental.pallas import tpu_sc as plsc

_LATENT = 16
_NSTEPS = 2

_NC, _NS = 2, 16
_NW = _NC * _NS

_B, _N, _D = 4, 100000, 20
_BPP = 2
_BLK = 1024
_NBLK = 98
_NPB = _NBLK * _BLK
_PRB = 13312
_VPB = _PRB * 8
_RPH = _BPP * _VPB
_ROWS_PER_W = _RPH // _NW
_CHUNK = 512
_NCHUNK = _ROWS_PER_W // _CHUNK
_IDX_W = 128
_IPC = _CHUNK // _IDX_W


def _tables_body(q_ref, w1_ref, t0_ref, t1_ref, t2_ref):
    outs = (t0_ref, t1_ref, t2_ref)
    for bb in range(_BPP):
        qb = q_ref[bb, :, :]
        for k in range(3):
            w1k = w1_ref[k * _D:(k + 1) * _D, :]
            m = lax.dot_general(qb, w1k, (((0,), (0,)), ((), ())),
                                preferred_element_type=jnp.float32)
            outs[k][bb, :, :] = jnp.concatenate(
                [m[s * 128:(s + 1) * 128, :] for s in range(8)], axis=1)


@functools.cache
def _make_tables():
    return pl.pallas_call(
        _tables_body,
        grid=(_NBLK,),
        in_specs=[
            pl.BlockSpec((_BPP, _D, _BLK), lambda i: (0, 0, i)),
            pl.BlockSpec((3 * _D, _LATENT), lambda i: (0, 0)),
        ],
        out_specs=[pl.BlockSpec((_BPP, 128, 128), lambda i: (0, i, 0))] * 3,
        out_shape=[jax.ShapeDtypeStruct((_BPP, _PRB, 128), jnp.float32)] * 3,
    )


def _gather_sum_body(p0, p1, p2, i0, i1, i2, out_hbm,
                     iv, rv, ov, sem):
    wid = lax.axis_index("s") * _NC + lax.axis_index("c")
    tabs = (p0, p1, p2)
    idxs = (i0, i1, i2)

    def issue(c, buf):
        irow = wid * (_ROWS_PER_W // _IDX_W) + c * _IPC
        for k in range(3):
            pltpu.sync_copy(idxs[k].at[pl.ds(irow, _IPC)], iv.at[buf, k])
        for j in range(_IPC):
            for k in range(3):
                pltpu.async_copy(tabs[k].at[iv.at[buf, k, j]],
                                 rv.at[buf, k, pl.ds(j * _IDX_W, _IDX_W)],
                                 sem)

    def drain(buf):
        for j in range(_IPC):
            for k in range(3):
                pltpu.make_async_copy(
                    tabs[k].at[iv.at[buf, k, j]],
                    rv.at[buf, k, pl.ds(j * _IDX_W, _IDX_W)], sem).wait()

    def sum_store(c, buf):
        def row_body(i, carry2):
            ov[i, :] = (rv[buf, 0, i, :] + rv[buf, 1, i, :]
                        + rv[buf, 2, i, :])
            return carry2

        lax.fori_loop(0, _CHUNK, row_body, 0, unroll=8)
        base = wid * _ROWS_PER_W + c * _CHUNK
        pltpu.sync_copy(ov, out_hbm.at[pl.ds(base, _CHUNK)])

    issue(0, 0)

    def outer(t, carry):
        c2 = 2 * t
        drain(0)
        issue(c2 + 1, 1)
        sum_store(c2, 0)
        drain(1)
        issue(c2 + 2, 0)
        sum_store(c2 + 1, 1)
        return carry

    lax.fori_loop(0, (_NCHUNK - 1) // 2, outer, 0)
    drain(0)
    sum_store(_NCHUNK - 1, 0)


@functools.cache
def _get_gather_sum():
    return pl.kernel(
        _gather_sum_body,
        out_type=jax.ShapeDtypeStruct((_RPH, _LATENT), jnp.float32),
        mesh=plsc.VectorSubcoreMesh(core_axis_name="c", subcore_axis_name="s",
                                    num_cores=_NC, num_subcores=_NS),
        scratch_types=[
            pltpu.VMEM((2, 3, _IPC, _IDX_W), jnp.int32),
            pltpu.VMEM((2, 3, _CHUNK, _LATENT), jnp.float32),
            pltpu.VMEM((_CHUNK, _LATENT), jnp.float32),
            pltpu.SemaphoreType.DMA,
        ],
        compiler_params=pltpu.CompilerParams(use_tc_tiling_on_sc=False),
    )


def _update_body(q_ref, g_ref, b1_ref, w2_ref, b2_ref, o_ref):
    for bb in range(_BPP):
        gb = g_ref[bb, :, :]
        mg = jnp.concatenate(
            [gb[:, s * 16:(s + 1) * 16] for s in range(8)], axis=0)
        h = jnp.tanh(mg + b1_ref[...])
        f = jnp.dot(h, w2_ref[...],
                    preferred_element_type=jnp.float32) + b2_ref[...]
        o_ref[bb, :, :] = q_ref[bb, :, :] + jnp.transpose(f, (1, 0))


@functools.cache
def _make_update():
    return pl.pallas_call(
        _update_body,
        grid=(_NBLK,),
        in_specs=[
            pl.BlockSpec((_BPP, _D, _BLK), lambda i: (0, 0, i)),
            pl.BlockSpec((_BPP, 128, 128), lambda i: (0, i, 0)),
            pl.BlockSpec((1, _LATENT), lambda i: (0, 0)),
            pl.BlockSpec((_LATENT, _D), lambda i: (0, 0)),
            pl.BlockSpec((1, _D), lambda i: (0, 0)),
        ],
        out_specs=pl.BlockSpec((_BPP, _D, _BLK), lambda i: (0, 0, i)),
        out_shape=jax.ShapeDtypeStruct((_BPP, _D, _N), jnp.float32),
    )


def kernel(inputs, W1, b1, W2, b2, neighbour_list):
    b, n, d = inputs.shape
    qhs = [jnp.transpose(inputs[h * _BPP:(h + 1) * _BPP], (0, 2, 1))
           for h in range(_B // _BPP)]

    padrows = jnp.broadcast_to(
        (jnp.arange(_NPB - n, dtype=jnp.int32) * 997 % n)[:, None],
        (_NPB - n, 3))
    nbr = jnp.concatenate([neighbour_list, padrows], axis=0)
    j = nbr.T
    q = j % _BLK
    vloc = (j // _BLK) * _BLK + (q % 128) * 8 + q // 128
    vloc = vloc.reshape(3, _NBLK, 8, 128).swapaxes(2, 3).reshape(3, _NPB)
    padv = jnp.broadcast_to(
        (jnp.arange(_VPB - _NPB, dtype=jnp.int32) * 1013 % _NPB)[None, :],
        (3, _VPB - _NPB))
    vloc = jnp.concatenate([vloc, padv], axis=1)
    idx = jnp.concatenate([vloc, vloc + _VPB], axis=1)
    idx = idx.reshape(3, _RPH // _IDX_W, _IDX_W)
    i0, i1, i2 = idx[0], idx[1], idx[2]

    w2p = jnp.pad(W2, ((0, 0), (0, d - _LATENT)))
    b1r = b1.reshape(1, _LATENT)
    b2r = jnp.pad(b2, (0, d - _LATENT)).reshape(1, d)

    tables = _make_tables()
    upd = _make_update()
    gather = _get_gather_sum()

    for _ in range(_NSTEPS):
        ts = [tables(qh, W1) for qh in qhs]
        gs = [gather(t0.reshape(_RPH, _LATENT), t1.reshape(_RPH, _LATENT),
                     t2.reshape(_RPH, _LATENT), i0, i1, i2)
              for (t0, t1, t2) in ts]
        qhs = [upd(qhs[h], gs[h].reshape(_BPP, _PRB, 128), b1r, w2p, b2r)
               for h in range(_B // _BPP)]
    return jnp.transpose(jnp.concatenate(qhs), (0, 2, 1))


# speedup vs baseline: 1.7145x; 1.2011x over previous
import functools

import jax
import jax.numpy as jnp
from jax import lax
from jax.experimental import pallas as pl
from jax.experimental.pallas import tpu as pltpu
from jax.experimental.pallas import tpu_sc as plsc

_LATENT = 16
_NSTEPS = 2

_NC, _NS = 2, 16
_NW = _NC * _NS

_B, _N, _D = 4, 100000, 20
_BLK = 1024
_NBLK = 98
_NPB = _NBLK * _BLK
_QCOLS = _NBLK * 128
_PRB = 13312
_VPB = _PRB * 8
_R_PAD = _B * _VPB
_ROWS_PER_W = _R_PAD // _NW
_CHUNK = 1024
_NCHUNK = _ROWS_PER_W // _CHUNK
_IDX_W = 128
_IPC = _CHUNK // _IDX_W


def _tables_body(ql_ref, qa_ref, kl_ref, ka_ref, t0_ref, t1_ref, t2_ref):
    outs = (t0_ref, t1_ref, t2_ref)
    for bb in range(_B):
        ql = ql_ref[bb]
        qa = qa_ref[bb]
        for k in range(3):
            tl = jnp.dot(ql, kl_ref[k],
                         preferred_element_type=jnp.float32)
            ta = lax.dot_general(qa, ka_ref[k], (((0,), (0,)), ((), ())),
                                 preferred_element_type=jnp.float32)
            outs[k][bb] = tl + ta


@functools.cache
def _make_tables():
    return pl.pallas_call(
        _tables_body,
        grid=(_NBLK,),
        in_specs=[
            pl.BlockSpec((_B, 128, 128), lambda i: (0, 0, i)),
            pl.BlockSpec((_B, 32, 128), lambda i: (0, 0, i)),
            pl.BlockSpec((3, 128, 128), lambda i: (0, 0, 0)),
            pl.BlockSpec((3, 32, 128), lambda i: (0, 0, 0)),
        ],
        out_specs=[pl.BlockSpec((_B, 128, 128), lambda i: (0, i, 0))] * 3,
        out_shape=[jax.ShapeDtypeStruct((_B, _PRB, 128), jnp.float32)] * 3,
    )


def _gather_sum_body(p0, p1, p2, i0, i1, i2, out_hbm,
                     iv, rv, ov, sem):
    wid = lax.axis_index("s") * _NC + lax.axis_index("c")
    tabs = (p0, p1, p2)
    idxs = (i0, i1, i2)

    def issue(c, buf):
        irow = wid * (_ROWS_PER_W // _IDX_W) + c * _IPC
        for k in range(3):
            pltpu.sync_copy(idxs[k].at[pl.ds(irow, _IPC)], iv.at[buf, k])
        for j in range(_IPC):
            for k in range(3):
                pltpu.async_copy(tabs[k].at[iv.at[buf, k, j]],
                                 rv.at[buf, k, pl.ds(j * _IDX_W, _IDX_W)],
                                 sem)

    def drain(buf):
        for j in range(_IPC):
            for k in range(3):
                pltpu.make_async_copy(
                    tabs[k].at[iv.at[buf, k, j]],
                    rv.at[buf, k, pl.ds(j * _IDX_W, _IDX_W)], sem).wait()

    def sum_store(c, buf):
        def row_body(i, carry2):
            ov[i, :] = (rv[buf, 0, i, :] + rv[buf, 1, i, :]
                        + rv[buf, 2, i, :])
            return carry2

        lax.fori_loop(0, _CHUNK, row_body, 0, unroll=8)
        base = wid * _ROWS_PER_W + c * _CHUNK
        pltpu.sync_copy(ov, out_hbm.at[pl.ds(base, _CHUNK)])

    issue(0, 0)

    def outer(t, carry):
        c2 = 2 * t
        drain(0)
        issue(c2 + 1, 1)
        sum_store(c2, 0)
        drain(1)
        issue(c2 + 2, 0)
        sum_store(c2 + 1, 1)
        return carry

    lax.fori_loop(0, (_NCHUNK - 1) // 2, outer, 0)
    drain(0)
    sum_store(_NCHUNK - 1, 0)


@functools.cache
def _get_gather_sum():
    return pl.kernel(
        _gather_sum_body,
        out_type=jax.ShapeDtypeStruct((_R_PAD, _LATENT), jnp.float32),
        mesh=plsc.VectorSubcoreMesh(core_axis_name="c", subcore_axis_name="s",
                                    num_cores=_NC, num_subcores=_NS),
        scratch_types=[
            pltpu.VMEM((2, 3, _IPC, _IDX_W), jnp.int32),
            pltpu.VMEM((2, 3, _CHUNK, _LATENT), jnp.float32),
            pltpu.VMEM((_CHUNK, _LATENT), jnp.float32),
            pltpu.SemaphoreType.DMA,
        ],
        compiler_params=pltpu.CompilerParams(use_tc_tiling_on_sc=False),
    )


def _update_body(ql_ref, g_ref, b1_ref, w2k_ref, b2_ref, o_ref):
    for bb in range(_B):
        hp = jnp.tanh(g_ref[bb] + b1_ref[...])
        fp = jnp.dot(hp, w2k_ref[...],
                     preferred_element_type=jnp.float32) + b2_ref[...]
        o_ref[bb] = ql_ref[bb] + fp


@functools.cache
def _make_update():
    return pl.pallas_call(
        _update_body,
        grid=(_NBLK,),
        in_specs=[
            pl.BlockSpec((_B, 128, 128), lambda i: (0, 0, i)),
            pl.BlockSpec((_B, 128, 128), lambda i: (0, i, 0)),
            pl.BlockSpec((1, 128), lambda i: (0, 0)),
            pl.BlockSpec((128, 128), lambda i: (0, 0)),
            pl.BlockSpec((1, 128), lambda i: (0, 0)),
        ],
        out_specs=pl.BlockSpec((_B, 128, 128), lambda i: (0, 0, i)),
        out_shape=jax.ShapeDtypeStruct((_B, 128, _QCOLS), jnp.float32),
    )


def kernel(inputs, W1, b1, W2, b2, neighbour_list):
    b, n, d = inputs.shape
    anc = d - _LATENT

    ypad = jnp.pad(inputs, ((0, 0), (0, _NPB - n), (0, 0)))
    yl = ypad[..., :_LATENT].reshape(_B, _NBLK, 8, 128, _LATENT)
    ql = yl.transpose(0, 3, 1, 2, 4).reshape(_B, 128, _QCOLS)
    ya = ypad[..., _LATENT:].reshape(_B, _NBLK, 8, 128, anc)
    qa = ya.transpose(0, 2, 4, 1, 3).reshape(_B, 8 * anc, _QCOLS)

    eye8 = jnp.eye(8, dtype=jnp.float32)
    w1r = W1.reshape(3, d, _LATENT)
    kl = jnp.stack([jnp.kron(eye8, w1r[k, :_LATENT, :]) for k in range(3)])
    ka = jnp.stack([jnp.kron(eye8, w1r[k, _LATENT:, :]) for k in range(3)])
    w2k = jnp.kron(eye8, W2)
    b1p = jnp.tile(b1, 8).reshape(1, 128)
    b2p = jnp.tile(b2, 8).reshape(1, 128)

    padrows = jnp.broadcast_to(
        (jnp.arange(_NPB - n, dtype=jnp.int32) * 997 % n)[:, None],
        (_NPB - n, 3))
    nbr = jnp.concatenate([neighbour_list, padrows], axis=0)
    j = nbr.T
    q = j % _BLK
    vloc = (j // _BLK) * _BLK + (q % 128) * 8 + q // 128
    vloc = vloc.reshape(3, _NBLK, 8, 128).swapaxes(2, 3).reshape(3, _NPB)
    padv = jnp.broadcast_to(
        (jnp.arange(_VPB - _NPB, dtype=jnp.int32) * 1013 % _NPB)[None, :],
        (3, _VPB - _NPB))
    vloc = jnp.concatenate([vloc, padv], axis=1)
    boffs = (jnp.arange(_B, dtype=jnp.int32) * _VPB)[None, :, None]
    idx = (vloc[:, None, :] + boffs).reshape(3, _R_PAD // _IDX_W, _IDX_W)
    i0, i1, i2 = idx[0], idx[1], idx[2]

    tables = _make_tables()
    upd = _make_update()
    gather = _get_gather_sum()

    for _ in range(_NSTEPS):
        t0, t1, t2 = tables(ql, qa, kl, ka)
        g = gather(t0.reshape(_R_PAD, _LATENT), t1.reshape(_R_PAD, _LATENT),
                   t2.reshape(_R_PAD, _LATENT), i0, i1, i2)
        ql = upd(ql, g.reshape(_B, _PRB, 128), b1p, w2k, b2p)

    out_l = ql.reshape(_B, 128, _NBLK, 8, _LATENT)
    out_l = out_l.transpose(0, 2, 3, 1, 4).reshape(_B, _NPB, _LATENT)[:, :n]
    return jnp.concatenate([out_l, inputs[..., _LATENT:]], axis=-1)


# speedup vs baseline: 1.8047x; 1.0526x over previous
import functools

import jax
import jax.numpy as jnp
from jax import lax
from jax.experimental import pallas as pl
from jax.experimental.pallas import tpu as pltpu
from jax.experimental.pallas import tpu_sc as plsc

_LATENT = 16
_NSTEPS = 2

_NC, _NS = 2, 16
_NW = _NC * _NS

_B, _N, _D = 4, 100000, 20
_BLK = 1024
_NBLK = 98
_NPB = _NBLK * _BLK
_QCOLS = _NBLK * 128
_PRB = 13312
_VPB = _PRB * 8
_R_PAD = _B * _VPB
_ROWS_PER_W = _R_PAD // _NW
_CHUNK = 1024
_NCHUNK = _ROWS_PER_W // _CHUNK
_IDX_W = 128
_IPC = _CHUNK // _IDX_W


def _tables_body(ql_ref, qa_ref, kl_ref, ka_ref, t0_ref, t1_ref, t2_ref):
    outs = (t0_ref, t1_ref, t2_ref)
    for bb in range(_B):
        ql = ql_ref[bb]
        qa = qa_ref[bb]
        for k in range(3):
            tl = jnp.dot(ql, kl_ref[k],
                         preferred_element_type=jnp.float32)
            ta = lax.dot_general(qa, ka_ref[k], (((0,), (0,)), ((), ())),
                                 preferred_element_type=jnp.float32)
            outs[k][bb] = tl + ta


@functools.cache
def _make_tables():
    return pl.pallas_call(
        _tables_body,
        grid=(_NBLK,),
        in_specs=[
            pl.BlockSpec((_B, 128, 128), lambda i: (0, 0, i)),
            pl.BlockSpec((_B, 32, 128), lambda i: (0, 0, i)),
            pl.BlockSpec((3, 128, 128), lambda i: (0, 0, 0)),
            pl.BlockSpec((3, 32, 128), lambda i: (0, 0, 0)),
        ],
        out_specs=[pl.BlockSpec((_B, 128, 128), lambda i: (0, i, 0))] * 3,
        out_shape=[jax.ShapeDtypeStruct((_B, _PRB, 128), jnp.float32)] * 3,
    )


def _gather_sum_body(p0, p1, p2, i0, i1, i2, out_hbm,
                     iv, rv, ov, sem):
    wid = lax.axis_index("s") * _NC + lax.axis_index("c")
    tabs = (p0, p1, p2)
    idxs = (i0, i1, i2)

    def issue(c, buf):
        irow = wid * (_ROWS_PER_W // _IDX_W) + c * _IPC
        for k in range(3):
            pltpu.sync_copy(idxs[k].at[pl.ds(irow, _IPC)], iv.at[buf, k])
        for j in range(_IPC):
            for k in range(3):
                pltpu.async_copy(tabs[k].at[iv.at[buf, k, j]],
                                 rv.at[buf, k, pl.ds(j * _IDX_W, _IDX_W)],
                                 sem)

    def drain(buf):
        for j in range(_IPC):
            for k in range(3):
                pltpu.make_async_copy(
                    tabs[k].at[iv.at[buf, k, j]],
                    rv.at[buf, k, pl.ds(j * _IDX_W, _IDX_W)], sem).wait()

    def sum_store(c, buf):
        def row_body(i, carry2):
            ov[i, :] = (rv[buf, 0, i, :] + rv[buf, 1, i, :]
                        + rv[buf, 2, i, :])
            return carry2

        lax.fori_loop(0, _CHUNK, row_body, 0, unroll=8)
        base = wid * _ROWS_PER_W + c * _CHUNK
        pltpu.sync_copy(ov, out_hbm.at[pl.ds(base, _CHUNK)])

    issue(0, 0)

    def outer(t, carry):
        c2 = 2 * t
        drain(0)
        issue(c2 + 1, 1)
        sum_store(c2, 0)
        drain(1)
        issue(c2 + 2, 0)
        sum_store(c2 + 1, 1)
        return carry

    lax.fori_loop(0, (_NCHUNK - 1) // 2, outer, 0)
    drain(0)
    sum_store(_NCHUNK - 1, 0)


@functools.cache
def _get_gather_sum():
    return pl.kernel(
        _gather_sum_body,
        out_type=jax.ShapeDtypeStruct((_R_PAD, _LATENT), jnp.float32),
        mesh=plsc.VectorSubcoreMesh(core_axis_name="c", subcore_axis_name="s",
                                    num_cores=_NC, num_subcores=_NS),
        scratch_types=[
            pltpu.VMEM((2, 3, _IPC, _IDX_W), jnp.int32),
            pltpu.VMEM((2, 3, _CHUNK, _LATENT), jnp.float32),
            pltpu.VMEM((_CHUNK, _LATENT), jnp.float32),
            pltpu.SemaphoreType.DMA,
        ],
        compiler_params=pltpu.CompilerParams(use_tc_tiling_on_sc=False),
    )


def _upd_tables_body(ql_ref, g_ref, qa_ref, b1_ref, w2k_ref, b2_ref,
                     kl_ref, ka_ref, o_ref, t0_ref, t1_ref, t2_ref):
    outs = (t0_ref, t1_ref, t2_ref)
    for bb in range(_B):
        hp = jnp.tanh(g_ref[bb] + b1_ref[...])
        fp = jnp.dot(hp, w2k_ref[...],
                     preferred_element_type=jnp.float32) + b2_ref[...]
        qln = ql_ref[bb] + fp
        o_ref[bb] = qln
        qa = qa_ref[bb]
        for k in range(3):
            tl = jnp.dot(qln, kl_ref[k],
                         preferred_element_type=jnp.float32)
            ta = lax.dot_general(qa, ka_ref[k], (((0,), (0,)), ((), ())),
                                 preferred_element_type=jnp.float32)
            outs[k][bb] = tl + ta


@functools.cache
def _make_upd_tables():
    return pl.pallas_call(
        _upd_tables_body,
        grid=(_NBLK,),
        in_specs=[
            pl.BlockSpec((_B, 128, 128), lambda i: (0, 0, i)),
            pl.BlockSpec((_B, 128, 128), lambda i: (0, i, 0)),
            pl.BlockSpec((_B, 32, 128), lambda i: (0, 0, i)),
            pl.BlockSpec((1, 128), lambda i: (0, 0)),
            pl.BlockSpec((128, 128), lambda i: (0, 0)),
            pl.BlockSpec((1, 128), lambda i: (0, 0)),
            pl.BlockSpec((3, 128, 128), lambda i: (0, 0, 0)),
            pl.BlockSpec((3, 32, 128), lambda i: (0, 0, 0)),
        ],
        out_specs=[pl.BlockSpec((_B, 128, 128), lambda i: (0, 0, i))]
        + [pl.BlockSpec((_B, 128, 128), lambda i: (0, i, 0))] * 3,
        out_shape=[jax.ShapeDtypeStruct((_B, 128, _QCOLS), jnp.float32)]
        + [jax.ShapeDtypeStruct((_B, _PRB, 128), jnp.float32)] * 3,
    )


def _update_body(ql_ref, g_ref, b1_ref, w2k_ref, b2_ref, o_ref):
    for bb in range(_B):
        hp = jnp.tanh(g_ref[bb] + b1_ref[...])
        fp = jnp.dot(hp, w2k_ref[...],
                     preferred_element_type=jnp.float32) + b2_ref[...]
        o_ref[bb] = ql_ref[bb] + fp


@functools.cache
def _make_update():
    return pl.pallas_call(
        _update_body,
        grid=(_NBLK,),
        in_specs=[
            pl.BlockSpec((_B, 128, 128), lambda i: (0, 0, i)),
            pl.BlockSpec((_B, 128, 128), lambda i: (0, i, 0)),
            pl.BlockSpec((1, 128), lambda i: (0, 0)),
            pl.BlockSpec((128, 128), lambda i: (0, 0)),
            pl.BlockSpec((1, 128), lambda i: (0, 0)),
        ],
        out_specs=pl.BlockSpec((_B, 128, 128), lambda i: (0, 0, i)),
        out_shape=jax.ShapeDtypeStruct((_B, 128, _QCOLS), jnp.float32),
    )


def kernel(inputs, W1, b1, W2, b2, neighbour_list):
    b, n, d = inputs.shape
    anc = d - _LATENT

    ypad = jnp.pad(inputs, ((0, 0), (0, _NPB - n), (0, 0)))
    yl = ypad[..., :_LATENT].reshape(_B, _NBLK, 8, 128, _LATENT)
    ql = yl.transpose(0, 3, 1, 2, 4).reshape(_B, 128, _QCOLS)
    ya = ypad[..., _LATENT:].reshape(_B, _NBLK, 8, 128, anc)
    qa = ya.transpose(0, 2, 4, 1, 3).reshape(_B, 8 * anc, _QCOLS)

    eye8 = jnp.eye(8, dtype=jnp.float32)
    w1r = W1.reshape(3, d, _LATENT)
    kl = jnp.stack([jnp.kron(eye8, w1r[k, :_LATENT, :]) for k in range(3)])
    ka = jnp.stack([jnp.kron(eye8, w1r[k, _LATENT:, :]) for k in range(3)])
    w2k = jnp.kron(eye8, W2)
    b1p = jnp.tile(b1, 8).reshape(1, 128)
    b2p = jnp.tile(b2, 8).reshape(1, 128)

    padrows = jnp.broadcast_to(
        (jnp.arange(_NPB - n, dtype=jnp.int32) * 997 % n)[:, None],
        (_NPB - n, 3))
    nbr = jnp.concatenate([neighbour_list, padrows], axis=0)
    j = nbr.T
    q = j % _BLK
    vloc = (j // _BLK) * _BLK + (q % 128) * 8 + q // 128
    vloc = vloc.reshape(3, _NBLK, 8, 128).swapaxes(2, 3).reshape(3, _NPB)
    padv = jnp.broadcast_to(
        (jnp.arange(_VPB - _NPB, dtype=jnp.int32) * 1013 % _NPB)[None, :],
        (3, _VPB - _NPB))
    vloc = jnp.concatenate([vloc, padv], axis=1)
    boffs = (jnp.arange(_B, dtype=jnp.int32) * _VPB)[None, :, None]
    idx = (vloc[:, None, :] + boffs).reshape(3, _R_PAD // _IDX_W, _IDX_W)
    i0, i1, i2 = idx[0], idx[1], idx[2]

    tables = _make_tables()
    upd = _make_update()
    updtab = _make_upd_tables()
    gather = _get_gather_sum()

    def do_gather(ts):
        t0, t1, t2 = ts
        return gather(t0.reshape(_R_PAD, _LATENT),
                      t1.reshape(_R_PAD, _LATENT),
                      t2.reshape(_R_PAD, _LATENT), i0, i1, i2)

    ts = tables(ql, qa, kl, ka)
    for step in range(_NSTEPS):
        g = do_gather(ts).reshape(_B, _PRB, 128)
        if step < _NSTEPS - 1:
            ql, *ts = updtab(ql, g, qa, b1p, w2k, b2p, kl, ka)
        else:
            ql = upd(ql, g, b1p, w2k, b2p)

    out_l = ql.reshape(_B, 128, _NBLK, 8, _LATENT)
    out_l = out_l.transpose(0, 2, 3, 1, 4).reshape(_B, _NPB, _LATENT)[:, :n]
    return jnp.concatenate([out_l, inputs[..., _LATENT:]], axis=-1)
